# Initial kernel scaffold; baseline (speedup 1.0000x reference)
import jax
import jax.numpy as jnp
from jax.experimental import pallas as pl


def kernel(x, edge_index, edge_attr, batch, params):
    raise NotImplementedError("write your pallas kernel here")


# speedup vs baseline: 7.9477x; 7.9477x over previous
import functools

import jax
import jax.numpy as jnp
from jax import lax
from jax.experimental import pallas as pl
from jax.experimental.pallas import tpu as pltpu
from jax.experimental.pallas import tpu_sc as plsc

N, E, DIN, DEDGE, H, G = 10000, 320000, 128, 16, 128, 256
NP = 10240
NC, NS = 2, 16
NW = NC * NS
CK = 128
EPT = 10112
EP = EPT * NW
NCHUNK = EPT // CK
BN = 2048
GRID_N = NP // BN
ROWS = NP // NS
EPS = 1e-16


def _mmt(a, w):
    return lax.dot_general(a, w, (((1,), (1,)), ((), ())),
                           preferred_element_type=jnp.float32)


def _mm(a, b):
    return lax.dot_general(a, b, (((1,), (0,)), ((), ())),
                           preferred_element_type=jnp.float32)


def _lrelu(v):
    return jnp.where(v > 0, v, 0.01 * v)


def _elu(v):
    return jnp.where(v > 0, v, jnp.exp(jnp.minimum(v, 0.0)) - 1.0)


def _gru_block(h, x_old, Wi, Wh, bi, bh):
    gi = _mmt(h, Wi) + bi
    gh = _mmt(x_old, Wh) + bh
    ir, iz, inn = gi[:, :H], gi[:, H:2 * H], gi[:, 2 * H:]
    hr, hz, hn = gh[:, :H], gh[:, H:2 * H], gh[:, 2 * H:]
    r = jax.nn.sigmoid(ir + hr)
    z = jax.nn.sigmoid(iz + hz)
    n = jnp.tanh(inn + r * hn)
    return (1.0 - z) * n + z * x_old


def _tc1_body(x_ref, wl_ref, bl_ref, wg1x_ref, attr_ref,
              x1_ref, xw1_ref, ar_ref):
    x1 = _lrelu(_mmt(x_ref[...], wl_ref[...]) + bl_ref[...])
    x1_ref[...] = x1
    xw1_ref[...] = _mmt(x1, wg1x_ref[...])
    ar_ref[...] = jnp.sum(x1 * attr_ref[...], axis=-1)


def _tc1(x_p, W_lin1, b_lin1, Wg1x, att_r):
    return pl.pallas_call(
        _tc1_body,
        grid=(GRID_N,),
        in_specs=[
            pl.BlockSpec((BN, DIN), lambda i: (i, 0)),
            pl.BlockSpec((H, DIN), lambda i: (0, 0)),
            pl.BlockSpec((H,), lambda i: (0,)),
            pl.BlockSpec((H, H), lambda i: (0, 0)),
            pl.BlockSpec((H,), lambda i: (0,)),
        ],
        out_specs=[
            pl.BlockSpec((BN, H), lambda i: (i, 0)),
            pl.BlockSpec((BN, H), lambda i: (i, 0)),
            pl.BlockSpec((BN,), lambda i: (i,)),
        ],
        out_shape=[
            jax.ShapeDtypeStruct((NP, H), jnp.float32),
            jax.ShapeDtypeStruct((NP, H), jnp.float32),
            jax.ShapeDtypeStruct((NP,), jnp.float32),
        ],
    )(x_p, W_lin1, b_lin1, Wg1x, att_r)


def _tc2_body(ea_ref, w_ref, out_ref):
    out_ref[...] = _mmt(ea_ref[...], w_ref[...])


def _tc2(ea_p, Wg1e):
    BE = 2048
    return pl.pallas_call(
        _tc2_body,
        grid=(EP // BE,),
        in_specs=[
            pl.BlockSpec((BE, DEDGE), lambda i: (i, 0)),
            pl.BlockSpec((H, DEDGE), lambda i: (0, 0)),
        ],
        out_specs=pl.BlockSpec((BE, H), lambda i: (i, 0)),
        out_shape=jax.ShapeDtypeStruct((EP, H), jnp.float32),
    )(ea_p, Wg1e)


def _tc_update_body(acc_ref, s_ref, xold_ref, wg2_ref, bg_ref,
                    wi_ref, wh_ref, bi_ref, bh_ref,
                    wa_ref, asrc_ref, adst_ref,
                    xn_ref, hx_ref, an_src_ref, an_dst_ref,
                    *, use_wg2):
    seg = (acc_ref[0] + acc_ref[1]) / (s_ref[0] + s_ref[1] + EPS)[:, None]
    if use_wg2:
        h = _elu(_mmt(seg, wg2_ref[...]) + bg_ref[...])
    else:
        h = _elu(seg + bg_ref[...])
    xn = jnp.maximum(
        _gru_block(h, xold_ref[...], wi_ref[...], wh_ref[...],
                   bi_ref[...], bh_ref[...]), 0.0)
    xn_ref[...] = xn
    hx = _mmt(xn, wa_ref[...])
    hx_ref[...] = hx
    an_src_ref[...] = jnp.sum(hx * asrc_ref[...], axis=-1)
    an_dst_ref[...] = jnp.sum(hx * adst_ref[...], axis=-1)


def _tc_update(acc, s, x_old, Wg2, bg, Wi, Wh, bi, bh, Wa, a_src, a_dst,
               use_wg2):
    body = functools.partial(_tc_update_body, use_wg2=use_wg2)
    return pl.pallas_call(
        body,
        grid=(GRID_N,),
        in_specs=[
            pl.BlockSpec((2, BN, H), lambda i: (0, i, 0)),
            pl.BlockSpec((2, BN), lambda i: (0, i)),
            pl.BlockSpec((BN, H), lambda i: (i, 0)),
            pl.BlockSpec((H, H), lambda i: (0, 0)),
            pl.BlockSpec((H,), lambda i: (0,)),
            pl.BlockSpec((3 * H, H), lambda i: (0, 0)),
            pl.BlockSpec((3 * H, H), lambda i: (0, 0)),
            pl.BlockSpec((3 * H,), lambda i: (0,)),
            pl.BlockSpec((3 * H,), lambda i: (0,)),
            pl.BlockSpec((H, H), lambda i: (0, 0)),
            pl.BlockSpec((H,), lambda i: (0,)),
            pl.BlockSpec((H,), lambda i: (0,)),
        ],
        out_specs=[
            pl.BlockSpec((BN, H), lambda i: (i, 0)),
            pl.BlockSpec((BN, H), lambda i: (i, 0)),
            pl.BlockSpec((BN,), lambda i: (i,)),
            pl.BlockSpec((BN,), lambda i: (i,)),
        ],
        out_shape=[
            jax.ShapeDtypeStruct((NP, H), jnp.float32),
            jax.ShapeDtypeStruct((NP, H), jnp.float32),
            jax.ShapeDtypeStruct((NP,), jnp.float32),
            jax.ShapeDtypeStruct((NP,), jnp.float32),
        ],
    )(acc, s, x_old, Wg2, bg, Wi, Wh, bi, bh, Wa, a_src, a_dst)


def _tc5_body(acc_ref, s_ref, xold_ref, bg_ref, wi_ref, wh_ref, bi_ref,
              bh_ref, wm_ref, asrcm_ref, batch_ref,
              hsrc_ref, an_src_ref, out0_ref):
    i = pl.program_id(0)
    seg = (acc_ref[0] + acc_ref[1]) / (s_ref[0] + s_ref[1] + EPS)[:, None]
    h = _elu(seg + bg_ref[...])
    xn = jnp.maximum(
        _gru_block(h, xold_ref[...], wi_ref[...], wh_ref[...],
                   bi_ref[...], bh_ref[...]), 0.0)
    hsrc = _mmt(xn, wm_ref[...])
    hsrc_ref[...] = hsrc
    an_src_ref[...] = jnp.sum(hsrc * asrcm_ref[...], axis=-1)
    at_blk = (batch_ref[...][None, :] ==
              lax.broadcasted_iota(jnp.int32, (G, BN), 0)).astype(jnp.float32)
    part = _mm(at_blk, xn)

    @pl.when(i == 0)
    def _():
        out0_ref[...] = part

    @pl.when(i > 0)
    def _():
        out0_ref[...] = out0_ref[...] + part

    @pl.when(i == GRID_N - 1)
    def _():
        out0_ref[...] = jnp.maximum(out0_ref[...], 0.0)


def _tc5(acc, s, x_old, bg, Wi, Wh, bi, bh, Wm, asrcm, batch_p):
    return pl.pallas_call(
        _tc5_body,
        grid=(GRID_N,),
        in_specs=[
            pl.BlockSpec((2, BN, H), lambda i: (0, i, 0)),
            pl.BlockSpec((2, BN), lambda i: (0, i)),
            pl.BlockSpec((BN, H), lambda i: (i, 0)),
            pl.BlockSpec((H,), lambda i: (0,)),
            pl.BlockSpec((3 * H, H), lambda i: (0, 0)),
            pl.BlockSpec((3 * H, H), lambda i: (0, 0)),
            pl.BlockSpec((3 * H,), lambda i: (0,)),
            pl.BlockSpec((3 * H,), lambda i: (0,)),
            pl.BlockSpec((H, H), lambda i: (0, 0)),
            pl.BlockSpec((H,), lambda i: (0,)),
            pl.BlockSpec((BN,), lambda i: (i,)),
        ],
        out_specs=[
            pl.BlockSpec((BN, H), lambda i: (i, 0)),
            pl.BlockSpec((BN,), lambda i: (i,)),
            pl.BlockSpec((G, H), lambda i: (0, 0)),
        ],
        out_shape=[
            jax.ShapeDtypeStruct((NP, H), jnp.float32),
            jax.ShapeDtypeStruct((NP,), jnp.float32),
            jax.ShapeDtypeStruct((G, H), jnp.float32),
        ],
    )(acc, s, x_old, bg, Wi, Wh, bi, bh, Wm, asrcm, batch_p)


def _mca_body(out_ref, wm_ref, adstm_ref, ansrc_ref, batch_ref,
              a_ref, bmax_ref):
    hdst = _mmt(out_ref[...], wm_ref[...])
    adst_g = jnp.sum(hdst * adstm_ref[...], axis=-1)
    a_blk = (batch_ref[...][:, None] ==
             lax.broadcasted_iota(jnp.int32, (BN, G), 1)).astype(jnp.float32)
    adst_n = _mm(a_blk, adst_g[:, None])[:, 0]
    a = _lrelu(ansrc_ref[...] + adst_n)
    a_ref[...] = a
    bmax_ref[...] = jnp.full((8, H), jnp.max(a), jnp.float32)


def _mca(out_g, Wm, adstm, ansrc, batch_p):
    return pl.pallas_call(
        _mca_body,
        grid=(GRID_N,),
        in_specs=[
            pl.BlockSpec((G, H), lambda i: (0, 0)),
            pl.BlockSpec((H, H), lambda i: (0, 0)),
            pl.BlockSpec((H,), lambda i: (0,)),
            pl.BlockSpec((BN,), lambda i: (i,)),
            pl.BlockSpec((BN,), lambda i: (i,)),
        ],
        out_specs=[
            pl.BlockSpec((BN,), lambda i: (i,)),
            pl.BlockSpec((8, H), lambda i: (i, 0)),
        ],
        out_shape=[
            jax.ShapeDtypeStruct((NP,), jnp.float32),
            jax.ShapeDtypeStruct((GRID_N * 8, H), jnp.float32),
        ],
    )(out_g, Wm, adstm, ansrc, batch_p)


def _mcb_body(a_ref, bmax_ref, hsrc_ref, batch_ref, out_ref,
              bm_ref, wi_ref, wh_ref, bi_ref, bh_ref,
              outn_ref, s_scr, u_scr):
    i = pl.program_id(0)
    gmax = jnp.max(bmax_ref[...])
    e = jnp.exp(a_ref[...] - gmax)
    at_blk = (batch_ref[...][None, :] ==
              lax.broadcasted_iota(jnp.int32, (G, BN), 0)).astype(jnp.float32)
    s_part = _mm(at_blk, e[:, None])
    u_part = _mm(at_blk, hsrc_ref[...] * e[:, None])

    @pl.when(i == 0)
    def _():
        s_scr[...] = s_part
        u_scr[...] = u_part

    @pl.when(i > 0)
    def _():
        s_scr[...] = s_scr[...] + s_part
        u_scr[...] = u_scr[...] + u_part

    @pl.when(i == GRID_N - 1)
    def _():
        h = _elu(u_scr[...] / (s_scr[...] + EPS) + bm_ref[...])
        outn_ref[...] = jnp.maximum(
            _gru_block(h, out_ref[...], wi_ref[...], wh_ref[...],
                       bi_ref[...], bh_ref[...]), 0.0)


def _mcb(a_n, bmax, hsrc, batch_p, out_g, bm, Wi, Wh, bi, bh):
    return pl.pallas_call(
        _mcb_body,
        grid=(GRID_N,),
        in_specs=[
            pl.BlockSpec((BN,), lambda i: (i,)),
            pl.BlockSpec((GRID_N * 8, H), lambda i: (0, 0)),
            pl.BlockSpec((BN, H), lambda i: (i, 0)),
            pl.BlockSpec((BN,), lambda i: (i,)),
            pl.BlockSpec((G, H), lambda i: (0, 0)),
            pl.BlockSpec((H,), lambda i: (0,)),
            pl.BlockSpec((3 * H, H), lambda i: (0, 0)),
            pl.BlockSpec((3 * H, H), lambda i: (0, 0)),
            pl.BlockSpec((3 * H,), lambda i: (0,)),
            pl.BlockSpec((3 * H,), lambda i: (0,)),
        ],
        out_specs=pl.BlockSpec((G, H), lambda i: (0, 0)),
        out_shape=jax.ShapeDtypeStruct((G, H), jnp.float32),
        scratch_shapes=[
            pltpu.VMEM((G, 1), jnp.float32),
            pltpu.VMEM((G, H), jnp.float32),
        ],
    )(a_n, bmax, hsrc, batch_p, out_g, bm, Wi, Wh, bi, bh)


def _tc6_body(out_ref, wl2_ref, bl2_ref, wm1_ref, bm1_ref, wm2_ref, bm2_ref,
              res_ref):
    o = jnp.maximum(_mmt(out_ref[...], wl2_ref[...]) + bl2_ref[...], 0.0)
    o = jnp.maximum(_mmt(o, wm1_ref[...]) + bm1_ref[...], 0.0)
    res_ref[...] = (jnp.sum(o * wm2_ref[...], axis=-1, keepdims=True)
                    + bm2_ref[...])


def _tc6(out_g, W_lin2, b_lin2, W_mlp1, b_mlp1, W_mlp2, b_mlp2):
    return pl.pallas_call(
        _tc6_body,
        out_shape=jax.ShapeDtypeStruct((G, 1), jnp.float32),
    )(out_g, W_lin2, b_lin2, W_mlp1, b_mlp1, W_mlp2, b_mlp2)


@functools.lru_cache(maxsize=None)
def _sc_mesh():
    return plsc.VectorSubcoreMesh(core_axis_name="c", subcore_axis_name="s",
                                  num_cores=NC, num_subcores=NS)


def _wid():
    return lax.axis_index("s") * NC + lax.axis_index("c")


def _bcast16(ref, j):
    return plsc.load_gather(ref, [jnp.full((16,), 0, jnp.int32) + j])


def _sc_gate_logits_body(src_hbm, dst_hbm, ew_hbm, xw1_hbm, ar_hbm, attl_hbm,
                         logit_hbm, tmax_hbm,
                         srcv, dstv, ewv, gatv, arv, attlv, alv, lgv, tmaxv,
                         sem):
    w = _wid()
    base = w * EPT
    pltpu.sync_copy(ar_hbm, arv)
    pltpu.sync_copy(attl_hbm, attlv)
    tmaxv[...] = jnp.full((16,), -jnp.inf, jnp.float32)

    def chunk(ci, _):
        cb = base + ci * CK
        pltpu.sync_copy(src_hbm.at[pl.ds(cb, CK)], srcv.at[0])
        pltpu.sync_copy(dst_hbm.at[pl.ds(cb, CK)], dstv.at[0])
        pltpu.sync_copy(ew_hbm.at[pl.ds(cb, CK), :], ewv)
        pltpu.async_copy(xw1_hbm.at[srcv.at[0]], gatv, sem).wait()

        def edge(j, _):
            acc = jnp.zeros((16,), jnp.float32)
            for v in range(8):
                sl = pl.ds(v * 16, 16)
                m = _lrelu(gatv[j, sl] + ewv[j, sl])
                acc = acc + m * attlv[sl]
            alv[pl.ds(j * 16, 16)] = acc
            return 0

        lax.fori_loop(0, CK, edge, 0)

        def group(g, _):
            lanes = (jnp.arange(16, dtype=jnp.int32) + g * 16) * 16
            al = jnp.zeros((16,), jnp.float32)
            for l in range(16):
                al = al + plsc.load_gather(
                    alv, [lanes + l])
            d = dstv[0, pl.ds(g * 16, 16)]
            ard = plsc.load_gather(arv, [d])
            lg = _lrelu(al + ard)
            lgv[pl.ds(g * 16, 16)] = lg
            tmaxv[...] = jnp.maximum(tmaxv[...], lg)
            return 0

        lax.fori_loop(0, CK // 16, group, 0)
        pltpu.sync_copy(lgv, logit_hbm.at[pl.ds(cb, CK)])
        return 0

    lax.fori_loop(0, NCHUNK, chunk, 0)
    pltpu.sync_copy(tmaxv, tmax_hbm.at[w])


def _sc_gate_logits(src_p, dst_p, ew, xw1, ar, att_l):
    return pl.kernel(
        _sc_gate_logits_body,
        out_type=[
            jax.ShapeDtypeStruct((EP,), jnp.float32),
            jax.ShapeDtypeStruct((NW, 16), jnp.float32),
        ],
        mesh=_sc_mesh(),
        scratch_types=[
            pltpu.VMEM((1, CK), jnp.int32),
            pltpu.VMEM((1, CK), jnp.int32),
            pltpu.VMEM((CK, H), jnp.float32),
            pltpu.VMEM((CK, H), jnp.float32),
            pltpu.VMEM((NP,), jnp.float32),
            pltpu.VMEM((H,), jnp.float32),
            pltpu.VMEM((CK * 16,), jnp.float32),
            pltpu.VMEM((CK,), jnp.float32),
            pltpu.VMEM((16,), jnp.float32),
            pltpu.SemaphoreType.DMA,
        ],
        compiler_params=pltpu.CompilerParams(needs_layout_passes=False),
    )(src_p, dst_p, ew, xw1, ar, att_l)


def _sc_gat_logits_body(src_hbm, dst_hbm, as_hbm, ad_hbm,
                        logit_hbm, tmax_hbm,
                        srcv, dstv, asv, adv, lgv, tmaxv):
    w = _wid()
    base = w * EPT
    pltpu.sync_copy(as_hbm, asv)
    pltpu.sync_copy(ad_hbm, adv)
    tmaxv[...] = jnp.full((16,), -jnp.inf, jnp.float32)

    def chunk(ci, _):
        cb = base + ci * CK
        pltpu.sync_copy(src_hbm.at[pl.ds(cb, CK)], srcv.at[0])
        pltpu.sync_copy(dst_hbm.at[pl.ds(cb, CK)], dstv.at[0])

        def group(g, _):
            sl = pl.ds(g * 16, 16)
            sa = plsc.load_gather(asv, [srcv[0, sl]])
            da = plsc.load_gather(adv, [dstv[0, sl]])
            lg = _lrelu(sa + da)
            lgv[sl] = lg
            tmaxv[...] = jnp.maximum(tmaxv[...], lg)
            return 0

        lax.fori_loop(0, CK // 16, group, 0)
        pltpu.sync_copy(lgv, logit_hbm.at[pl.ds(cb, CK)])
        return 0

    lax.fori_loop(0, NCHUNK, chunk, 0)
    pltpu.sync_copy(tmaxv, tmax_hbm.at[w])


def _sc_gat_logits(src_p, dst_p, asrc_n, adst_n):
    return pl.kernel(
        _sc_gat_logits_body,
        out_type=[
            jax.ShapeDtypeStruct((EP,), jnp.float32),
            jax.ShapeDtypeStruct((NW, 16), jnp.float32),
        ],
        mesh=_sc_mesh(),
        scratch_types=[
            pltpu.VMEM((1, CK), jnp.int32),
            pltpu.VMEM((1, CK), jnp.int32),
            pltpu.VMEM((NP,), jnp.float32),
            pltpu.VMEM((NP,), jnp.float32),
            pltpu.VMEM((CK,), jnp.float32),
            pltpu.VMEM((16,), jnp.float32),
        ],
        compiler_params=pltpu.CompilerParams(needs_layout_passes=False),
    )(src_p, dst_p, asrc_n, adst_n)


def _sc_scatter_body(src_hbm, dst_hbm, tab_hbm, ew_hbm, lg_hbm, tmax_hbm,
                     zrow_hbm, zvec_hbm, acc_hbm, s_hbm,
                     srcv, dstv, ewv, gatv, lgv, ebuf, tmaxs, gmaxv,
                     acc_sh, s_sh, sem, *, gate):
    c = lax.axis_index("c")
    sid = lax.axis_index("s")
    base = (sid * NC + c) * EPT

    pltpu.sync_copy(zrow_hbm, acc_sh.at[pl.ds(sid * ROWS, ROWS), :])
    pltpu.sync_copy(zvec_hbm, s_sh.at[pl.ds(sid * ROWS, ROWS)])
    pltpu.sync_copy(tmax_hbm, tmaxs)
    gm = jnp.full((16,), -jnp.inf, jnp.float32)
    for t in range(NW):
        gm = jnp.maximum(gm, tmaxs[t, :])
    gmaxv[...] = jnp.full((16,), 0.0, jnp.float32) + jnp.max(gm)
    plsc.subcore_barrier()

    def chunk(ci, _):
        cb = base + ci * CK
        pltpu.sync_copy(src_hbm.at[pl.ds(cb, CK)], srcv.at[0])
        pltpu.sync_copy(dst_hbm.at[pl.ds(cb, CK)], dstv.at[0])
        pltpu.sync_copy(lg_hbm.at[pl.ds(cb, CK)], lgv)
        if gate:
            pltpu.sync_copy(ew_hbm.at[pl.ds(cb, CK), :], ewv)
        pltpu.async_copy(tab_hbm.at[srcv.at[0]], gatv, sem).wait()
        gmax = gmaxv[...]

        def group(g, _):
            sl = pl.ds(g * 16, 16)
            ebuf[sl] = jnp.exp(lgv[sl] - gmax)
            return 0

        lax.fori_loop(0, CK // 16, group, 0)

        def edge(j, _):
            eb = _bcast16(ebuf, j)
            for v in range(8):
                sl = pl.ds(v * 16, 16)
                if gate:
                    row = _lrelu(gatv[j, sl] + ewv[j, sl])
                else:
                    row = gatv[j, sl]
                gatv[j, sl] = row * eb
            return 0

        lax.fori_loop(0, CK, edge, 0)
        pltpu.sync_copy(gatv, acc_sh.at[dstv.at[0]], add=True)
        pltpu.sync_copy(ebuf, s_sh.at[dstv.at[0]], add=True)
        return 0

    lax.fori_loop(0, NCHUNK, chunk, 0)
    plsc.subcore_barrier()
    pltpu.sync_copy(acc_sh.at[pl.ds(sid * ROWS, ROWS), :],
                    acc_hbm.at[c, pl.ds(sid * ROWS, ROWS), :])
    pltpu.sync_copy(s_sh.at[pl.ds(sid * ROWS, ROWS)],
                    s_hbm.at[c, pl.ds(sid * ROWS, ROWS)])


def _sc_scatter(src_p, dst_p, table, ew, logit, tmax, zrow, zvec, gate):
    body = functools.partial(_sc_scatter_body, gate=gate)
    return pl.kernel(
        body,
        out_type=[
            jax.ShapeDtypeStruct((NC, NP, H), jnp.float32),
            jax.ShapeDtypeStruct((NC, NP), jnp.float32),
        ],
        mesh=_sc_mesh(),
        scratch_types=[
            pltpu.VMEM((1, CK), jnp.int32),
            pltpu.VMEM((1, CK), jnp.int32),
            pltpu.VMEM((CK, H), jnp.float32),
            pltpu.VMEM((CK, H), jnp.float32),
            pltpu.VMEM((CK,), jnp.float32),
            pltpu.VMEM((CK,), jnp.float32),
            pltpu.VMEM((NW, 16), jnp.float32),
            pltpu.VMEM((16,), jnp.float32),
            pltpu.VMEM_SHARED((NP, H), jnp.float32),
            pltpu.VMEM_SHARED((NP,), jnp.float32),
            pltpu.SemaphoreType.DMA,
        ],
        compiler_params=pltpu.CompilerParams(needs_layout_passes=False),
    )(src_p, dst_p, table, ew, logit, tmax, zrow, zvec)


def kernel(x, edge_index, edge_attr, batch, params):
    p = params
    x_p = jnp.zeros((NP, DIN), jnp.float32).at[:N].set(x)
    src_p = jnp.full((EP,), N, jnp.int32).at[:E].set(edge_index[0])
    dst_p = jnp.full((EP,), N, jnp.int32).at[:E].set(edge_index[1])
    ea_p = jnp.zeros((EP, DEDGE), jnp.float32).at[:E].set(edge_attr)
    batch_p = jnp.full((NP,), G, jnp.int32).at[:N].set(batch)
    zrow = jnp.zeros((ROWS, H), jnp.float32)
    zvec = jnp.zeros((ROWS,), jnp.float32)
    Wg1x = p['Wg1'][:, :H]
    Wg1e = p['Wg1'][:, H:]

    x1, xw1, ar = _tc1(x_p, p['W_lin1'], p['b_lin1'], Wg1x, p['att_r'])
    ew = _tc2(ea_p, Wg1e)

    logit, tmax = _sc_gate_logits(src_p, dst_p, ew, xw1, ar, p['att_l'])
    acc, s = _sc_scatter(src_p, dst_p, xw1, ew, logit, tmax, zrow, zvec, True)
    x2, hx1, as1, ad1 = _tc_update(
        acc, s, x1, p['Wg2'], p['b_gate'], p['Wi0'], p['Wh0'], p['bi0'],
        p['bh0'], p['Wa1'], p['asrc1'], p['adst1'], True)

    logit, tmax = _sc_gat_logits(src_p, dst_p, as1, ad1)
    acc, s = _sc_scatter(src_p, dst_p, hx1, ew, logit, tmax, zrow, zvec,
                         False)
    x3, hx2, as2, ad2 = _tc_update(
        acc, s, x2, p['Wg2'], p['ba1'], p['Wi1'], p['Wh1'], p['bi1'],
        p['bh1'], p['Wa2'], p['asrc2'], p['adst2'], False)

    logit, tmax = _sc_gat_logits(src_p, dst_p, as2, ad2)
    acc, s = _sc_scatter(src_p, dst_p, hx2, ew, logit, tmax, zrow, zvec,
                         False)

    hsrc, ansrc, out_g = _tc5(
        acc, s, x3, p['ba2'], p['Wi2'], p['Wh2'], p['bi2'], p['bh2'],
        p['Wm'], p['asrcm'], batch_p)

    for _ in range(3):
        a_n, bmax = _mca(out_g, p['Wm'], p['adstm'], ansrc, batch_p)
        out_g = _mcb(a_n, bmax, hsrc, batch_p, out_g, p['bm'],
                     p['Wim'], p['Whm'], p['bim'], p['bhm'])

    return _tc6(out_g, p['W_lin2'], p['b_lin2'], p['W_mlp1'], p['b_mlp1'],
                p['W_mlp2'], p['b_mlp2'])


# speedup vs baseline: 9.4520x; 1.1893x over previous
import functools

import jax
import jax.numpy as jnp
from jax import lax
from jax.experimental import pallas as pl
from jax.experimental.pallas import tpu as pltpu
from jax.experimental.pallas import tpu_sc as plsc

N, E, DIN, DEDGE, H, G = 10000, 320000, 128, 16, 128, 256
NP = 10240
NC, NS = 2, 16
NW = NC * NS
CK = 128
EPT = 10112
EP = EPT * NW
NCHUNK = EPT // CK
BN = 2048
GRID_N = NP // BN
ROWS = NP // NS
EPS = 1e-16


def _mmt(a, w):
    return lax.dot_general(a, w, (((1,), (1,)), ((), ())),
                           preferred_element_type=jnp.float32)


def _mm(a, b):
    return lax.dot_general(a, b, (((1,), (0,)), ((), ())),
                           preferred_element_type=jnp.float32)


def _lrelu(v):
    return jnp.where(v > 0, v, 0.01 * v)


def _elu(v):
    return jnp.where(v > 0, v, jnp.exp(jnp.minimum(v, 0.0)) - 1.0)


def _gru_block(h, x_old, Wi, Wh, bi, bh):
    gi = _mmt(h, Wi) + bi
    gh = _mmt(x_old, Wh) + bh
    ir, iz, inn = gi[:, :H], gi[:, H:2 * H], gi[:, 2 * H:]
    hr, hz, hn = gh[:, :H], gh[:, H:2 * H], gh[:, 2 * H:]
    r = jax.nn.sigmoid(ir + hr)
    z = jax.nn.sigmoid(iz + hz)
    n = jnp.tanh(inn + r * hn)
    return (1.0 - z) * n + z * x_old


def _tc1_body(x_ref, wl_ref, bl_ref, wg1x_ref, attr_ref,
              x1_ref, xw1_ref, ar_ref):
    x1 = _lrelu(_mmt(x_ref[...], wl_ref[...]) + bl_ref[...])
    x1_ref[...] = x1
    xw1_ref[...] = _mmt(x1, wg1x_ref[...])
    ar_ref[...] = jnp.sum(x1 * attr_ref[...], axis=-1)


def _tc1(x_p, W_lin1, b_lin1, Wg1x, att_r):
    return pl.pallas_call(
        _tc1_body,
        grid=(GRID_N,),
        in_specs=[
            pl.BlockSpec((BN, DIN), lambda i: (i, 0)),
            pl.BlockSpec((H, DIN), lambda i: (0, 0)),
            pl.BlockSpec((H,), lambda i: (0,)),
            pl.BlockSpec((H, H), lambda i: (0, 0)),
            pl.BlockSpec((H,), lambda i: (0,)),
        ],
        out_specs=[
            pl.BlockSpec((BN, H), lambda i: (i, 0)),
            pl.BlockSpec((BN, H), lambda i: (i, 0)),
            pl.BlockSpec((BN,), lambda i: (i,)),
        ],
        out_shape=[
            jax.ShapeDtypeStruct((NP, H), jnp.float32),
            jax.ShapeDtypeStruct((NP, H), jnp.float32),
            jax.ShapeDtypeStruct((NP,), jnp.float32),
        ],
    )(x_p, W_lin1, b_lin1, Wg1x, att_r)


def _tc2_body(ea_ref, w_ref, out_ref):
    out_ref[...] = _mmt(ea_ref[...], w_ref[...])


def _tc2(ea_p, Wg1e):
    BE = 2048
    return pl.pallas_call(
        _tc2_body,
        grid=(EP // BE,),
        in_specs=[
            pl.BlockSpec((BE, DEDGE), lambda i: (i, 0)),
            pl.BlockSpec((H, DEDGE), lambda i: (0, 0)),
        ],
        out_specs=pl.BlockSpec((BE, H), lambda i: (i, 0)),
        out_shape=jax.ShapeDtypeStruct((EP, H), jnp.float32),
    )(ea_p, Wg1e)


def _tc_update_body(acc_ref, s_ref, xold_ref, wg2_ref, bg_ref,
                    wi_ref, wh_ref, bi_ref, bh_ref,
                    wa_ref, asrc_ref, adst_ref,
                    xn_ref, hx_ref, an_src_ref, an_dst_ref,
                    *, use_wg2):
    seg = (acc_ref[0] + acc_ref[1]) / (s_ref[0] + s_ref[1] + EPS)[:, None]
    if use_wg2:
        h = _elu(_mmt(seg, wg2_ref[...]) + bg_ref[...])
    else:
        h = _elu(seg + bg_ref[...])
    xn = jnp.maximum(
        _gru_block(h, xold_ref[...], wi_ref[...], wh_ref[...],
                   bi_ref[...], bh_ref[...]), 0.0)
    xn_ref[...] = xn
    hx = _mmt(xn, wa_ref[...])
    hx_ref[...] = hx
    an_src_ref[...] = jnp.sum(hx * asrc_ref[...], axis=-1)
    an_dst_ref[...] = jnp.sum(hx * adst_ref[...], axis=-1)


def _tc_update(acc, s, x_old, Wg2, bg, Wi, Wh, bi, bh, Wa, a_src, a_dst,
               use_wg2):
    body = functools.partial(_tc_update_body, use_wg2=use_wg2)
    return pl.pallas_call(
        body,
        grid=(GRID_N,),
        in_specs=[
            pl.BlockSpec((2, BN, H), lambda i: (0, i, 0)),
            pl.BlockSpec((2, BN), lambda i: (0, i)),
            pl.BlockSpec((BN, H), lambda i: (i, 0)),
            pl.BlockSpec((H, H), lambda i: (0, 0)),
            pl.BlockSpec((H,), lambda i: (0,)),
            pl.BlockSpec((3 * H, H), lambda i: (0, 0)),
            pl.BlockSpec((3 * H, H), lambda i: (0, 0)),
            pl.BlockSpec((3 * H,), lambda i: (0,)),
            pl.BlockSpec((3 * H,), lambda i: (0,)),
            pl.BlockSpec((H, H), lambda i: (0, 0)),
            pl.BlockSpec((H,), lambda i: (0,)),
            pl.BlockSpec((H,), lambda i: (0,)),
        ],
        out_specs=[
            pl.BlockSpec((BN, H), lambda i: (i, 0)),
            pl.BlockSpec((BN, H), lambda i: (i, 0)),
            pl.BlockSpec((BN,), lambda i: (i,)),
            pl.BlockSpec((BN,), lambda i: (i,)),
        ],
        out_shape=[
            jax.ShapeDtypeStruct((NP, H), jnp.float32),
            jax.ShapeDtypeStruct((NP, H), jnp.float32),
            jax.ShapeDtypeStruct((NP,), jnp.float32),
            jax.ShapeDtypeStruct((NP,), jnp.float32),
        ],
    )(acc, s, x_old, Wg2, bg, Wi, Wh, bi, bh, Wa, a_src, a_dst)


def _tc5_body(acc_ref, s_ref, xold_ref, bg_ref, wi_ref, wh_ref, bi_ref,
              bh_ref, wm_ref, asrcm_ref, batch_ref,
              hsrc_ref, an_src_ref, out0_ref):
    i = pl.program_id(0)
    seg = (acc_ref[0] + acc_ref[1]) / (s_ref[0] + s_ref[1] + EPS)[:, None]
    h = _elu(seg + bg_ref[...])
    xn = jnp.maximum(
        _gru_block(h, xold_ref[...], wi_ref[...], wh_ref[...],
                   bi_ref[...], bh_ref[...]), 0.0)
    hsrc = _mmt(xn, wm_ref[...])
    hsrc_ref[...] = hsrc
    an_src_ref[...] = jnp.sum(hsrc * asrcm_ref[...], axis=-1)
    at_blk = (batch_ref[...][None, :] ==
              lax.broadcasted_iota(jnp.int32, (G, BN), 0)).astype(jnp.float32)
    part = _mm(at_blk, xn)

    @pl.when(i == 0)
    def _():
        out0_ref[...] = part

    @pl.when(i > 0)
    def _():
        out0_ref[...] = out0_ref[...] + part

    @pl.when(i == GRID_N - 1)
    def _():
        out0_ref[...] = jnp.maximum(out0_ref[...], 0.0)


def _tc5(acc, s, x_old, bg, Wi, Wh, bi, bh, Wm, asrcm, batch_p):
    return pl.pallas_call(
        _tc5_body,
        grid=(GRID_N,),
        in_specs=[
            pl.BlockSpec((2, BN, H), lambda i: (0, i, 0)),
            pl.BlockSpec((2, BN), lambda i: (0, i)),
            pl.BlockSpec((BN, H), lambda i: (i, 0)),
            pl.BlockSpec((H,), lambda i: (0,)),
            pl.BlockSpec((3 * H, H), lambda i: (0, 0)),
            pl.BlockSpec((3 * H, H), lambda i: (0, 0)),
            pl.BlockSpec((3 * H,), lambda i: (0,)),
            pl.BlockSpec((3 * H,), lambda i: (0,)),
            pl.BlockSpec((H, H), lambda i: (0, 0)),
            pl.BlockSpec((H,), lambda i: (0,)),
            pl.BlockSpec((BN,), lambda i: (i,)),
        ],
        out_specs=[
            pl.BlockSpec((BN, H), lambda i: (i, 0)),
            pl.BlockSpec((BN,), lambda i: (i,)),
            pl.BlockSpec((G, H), lambda i: (0, 0)),
        ],
        out_shape=[
            jax.ShapeDtypeStruct((NP, H), jnp.float32),
            jax.ShapeDtypeStruct((NP,), jnp.float32),
            jax.ShapeDtypeStruct((G, H), jnp.float32),
        ],
    )(acc, s, x_old, bg, Wi, Wh, bi, bh, Wm, asrcm, batch_p)


def _mc_iter_body(hsrc_ref, ansrc_ref, batch_ref, out_ref,
                  wm_ref, adstm_ref, bm_ref, wi_ref, wh_ref, bi_ref, bh_ref,
                  outn_ref, m_scr, s_scr, u_scr):
    i = pl.program_id(0)
    hdst = _mmt(out_ref[...], wm_ref[...])
    adst_g = jnp.sum(hdst * adstm_ref[...], axis=-1)
    a_blk = (batch_ref[...][:, None] ==
             lax.broadcasted_iota(jnp.int32, (BN, G), 1)).astype(jnp.float32)
    a = _lrelu(ansrc_ref[...] + _mm(a_blk, adst_g[:, None])[:, 0])
    bmax = jnp.max(a)

    @pl.when(i == 0)
    def _():
        e = jnp.exp(a - bmax)
        at_blk = (batch_ref[...][None, :] == lax.broadcasted_iota(
            jnp.int32, (G, BN), 0)).astype(jnp.float32)
        m_scr[0, 0] = bmax
        s_scr[...] = _mm(at_blk, e[:, None])
        u_scr[...] = _mm(at_blk, hsrc_ref[...] * e[:, None])

    @pl.when(i > 0)
    def _():
        m_old = m_scr[0, 0]
        m_new = jnp.maximum(m_old, bmax)
        scale = jnp.exp(m_old - m_new)
        e = jnp.exp(a - m_new)
        at_blk = (batch_ref[...][None, :] == lax.broadcasted_iota(
            jnp.int32, (G, BN), 0)).astype(jnp.float32)
        m_scr[0, 0] = m_new
        s_scr[...] = s_scr[...] * scale + _mm(at_blk, e[:, None])
        u_scr[...] = u_scr[...] * scale + _mm(at_blk, hsrc_ref[...]
                                              * e[:, None])

    @pl.when(i == GRID_N - 1)
    def _():
        h = _elu(u_scr[...] / (s_scr[...] + EPS) + bm_ref[...])
        outn_ref[...] = jnp.maximum(
            _gru_block(h, out_ref[...], wi_ref[...], wh_ref[...],
                       bi_ref[...], bh_ref[...]), 0.0)


def _mc_iter(hsrc, ansrc, batch_p, out_g, Wm, adstm, bm, Wi, Wh, bi, bh):
    return pl.pallas_call(
        _mc_iter_body,
        grid=(GRID_N,),
        in_specs=[
            pl.BlockSpec((BN, H), lambda i: (i, 0)),
            pl.BlockSpec((BN,), lambda i: (i,)),
            pl.BlockSpec((BN,), lambda i: (i,)),
            pl.BlockSpec((G, H), lambda i: (0, 0)),
            pl.BlockSpec((H, H), lambda i: (0, 0)),
            pl.BlockSpec((H,), lambda i: (0,)),
            pl.BlockSpec((H,), lambda i: (0,)),
            pl.BlockSpec((3 * H, H), lambda i: (0, 0)),
            pl.BlockSpec((3 * H, H), lambda i: (0, 0)),
            pl.BlockSpec((3 * H,), lambda i: (0,)),
            pl.BlockSpec((3 * H,), lambda i: (0,)),
        ],
        out_specs=pl.BlockSpec((G, H), lambda i: (0, 0)),
        out_shape=jax.ShapeDtypeStruct((G, H), jnp.float32),
        scratch_shapes=[
            pltpu.SMEM((1, 1), jnp.float32),
            pltpu.VMEM((G, 1), jnp.float32),
            pltpu.VMEM((G, H), jnp.float32),
        ],
    )(hsrc, ansrc, batch_p, out_g, Wm, adstm, bm, Wi, Wh, bi, bh)


def _tc6_body(out_ref, wl2_ref, bl2_ref, wm1_ref, bm1_ref, wm2_ref, bm2_ref,
              res_ref):
    o = jnp.maximum(_mmt(out_ref[...], wl2_ref[...]) + bl2_ref[...], 0.0)
    o = jnp.maximum(_mmt(o, wm1_ref[...]) + bm1_ref[...], 0.0)
    res_ref[...] = (jnp.sum(o * wm2_ref[...], axis=-1, keepdims=True)
                    + bm2_ref[...])


def _tc6(out_g, W_lin2, b_lin2, W_mlp1, b_mlp1, W_mlp2, b_mlp2):
    return pl.pallas_call(
        _tc6_body,
        out_shape=jax.ShapeDtypeStruct((G, 1), jnp.float32),
    )(out_g, W_lin2, b_lin2, W_mlp1, b_mlp1, W_mlp2, b_mlp2)


@functools.lru_cache(maxsize=None)
def _sc_mesh():
    return plsc.VectorSubcoreMesh(core_axis_name="c", subcore_axis_name="s",
                                  num_cores=NC, num_subcores=NS)


def _wid():
    return lax.axis_index("s") * NC + lax.axis_index("c")


def _bcast16(ref, j):
    return plsc.load_gather(ref, [jnp.full((16,), 0, jnp.int32) + j])


def _sc_gate_logits_body(src_hbm, dst_hbm, ew_hbm, xw1_hbm, ar_hbm, attl_hbm,
                         logit_hbm, tmax_hbm,
                         srcv0, srcv1, dstv0, dstv1, lgv0, lgv1,
                         ewv0, ewv1, gatv0, gatv1, alv, arv, attlv, tmaxv,
                         sin0, sin1, sg0, sg1, so0, so1):
    w = _wid()
    base = w * EPT
    srcv = (srcv0, srcv1)
    dstv = (dstv0, dstv1)
    lgv = (lgv0, lgv1)
    ewv = (ewv0, ewv1)
    gatv = (gatv0, gatv1)
    sin = (sin0, sin1)
    sg = (sg0, sg1)
    so = (so0, so1)
    pltpu.sync_copy(ar_hbm, arv)
    pltpu.sync_copy(attl_hbm, attlv)
    tmaxv[...] = jnp.full((16,), -jnp.inf, jnp.float32)

    def fire_in(k2, b):
        cb = base + k2 * CK2
        pltpu.async_copy(src_hbm.at[pl.ds(cb, CK2)], srcv[b].at[0], sin[b])
        pltpu.async_copy(dst_hbm.at[pl.ds(cb, CK2)], dstv[b].at[0], sin[b])
        pltpu.async_copy(ew_hbm.at[pl.ds(cb, CK2), :], ewv[b], sin[b])

    def wait_in(k2, b):
        cb = base + k2 * CK2
        pltpu.make_async_copy(src_hbm.at[pl.ds(cb, CK2)], srcv[b].at[0],
                              sin[b]).wait()
        pltpu.make_async_copy(dst_hbm.at[pl.ds(cb, CK2)], dstv[b].at[0],
                              sin[b]).wait()
        pltpu.make_async_copy(ew_hbm.at[pl.ds(cb, CK2), :], ewv[b],
                              sin[b]).wait()

    def wait_out(k2, b):
        cb = base + k2 * CK2
        pltpu.make_async_copy(lgv[b], logit_hbm.at[pl.ds(cb, CK2)],
                              so[b]).wait()

    fire_in(0, 0)

    def pair(gi, _):
        for b in (0, 1):
            k2 = gi * 2 + b
            wait_in(k2, b)

            @pl.when(k2 >= 2)
            def _():
                wait_out(k2 - 2, b)

            pltpu.async_copy(xw1_hbm.at[srcv[b].at[0]], gatv[b], sg[b])

            @pl.when(k2 + 1 < NCHUNK2)
            def _():
                fire_in(k2 + 1, 1 - b)

            pltpu.make_async_copy(xw1_hbm.at[srcv[b].at[0]], gatv[b],
                                  sg[b]).wait()

            def edge(j, _):
                acc = jnp.zeros((16,), jnp.float32)
                for v in range(8):
                    sl = pl.ds(v * 16, 16)
                    m = _lrelu(gatv[b][j, sl] + ewv[b][j, sl])
                    acc = acc + m * attlv[sl]
                alv[pl.ds(j * 16, 16)] = acc
                return 0

            lax.fori_loop(0, CK2, edge, 0)

            def group(g, _):
                lanes = (jnp.arange(16, dtype=jnp.int32) + g * 16) * 16
                al = jnp.zeros((16,), jnp.float32)
                for l in range(16):
                    al = al + plsc.load_gather(alv, [lanes + l])
                d = dstv[b][0, pl.ds(g * 16, 16)]
                ard = plsc.load_gather(arv, [d])
                lg = _lrelu(al + ard)
                lgv[b][pl.ds(g * 16, 16)] = lg
                tmaxv[...] = jnp.maximum(tmaxv[...], lg)
                return 0

            lax.fori_loop(0, CK2 // 16, group, 0)
            cb = base + k2 * CK2
            pltpu.async_copy(lgv[b], logit_hbm.at[pl.ds(cb, CK2)], so[b])
        return 0

    lax.fori_loop(0, NCHUNK2 // 2, pair, 0)
    wait_out(NCHUNK2 - 2, 0)
    wait_out(NCHUNK2 - 1, 1)
    pltpu.sync_copy(tmaxv, tmax_hbm.at[w])


def _sc_gate_logits(src_p, dst_p, ew, xw1, ar, att_l):
    return pl.kernel(
        _sc_gate_logits_body,
        out_type=[
            jax.ShapeDtypeStruct((EP,), jnp.float32),
            jax.ShapeDtypeStruct((NW, 16), jnp.float32),
        ],
        mesh=_sc_mesh(),
        scratch_types=(
            [pltpu.VMEM((1, CK2), jnp.int32) for _ in range(4)]
            + [pltpu.VMEM((CK2,), jnp.float32) for _ in range(2)]
            + [pltpu.VMEM((CK2, H), jnp.float32) for _ in range(4)]
            + [pltpu.VMEM((CK2 * 16,), jnp.float32),
               pltpu.VMEM((NP,), jnp.float32),
               pltpu.VMEM((H,), jnp.float32),
               pltpu.VMEM((16,), jnp.float32)]
            + [pltpu.SemaphoreType.DMA for _ in range(6)]
        ),
        compiler_params=pltpu.CompilerParams(needs_layout_passes=False),
    )(src_p, dst_p, ew, xw1, ar, att_l)


def _sc_gat_logits_body(src_hbm, dst_hbm, as_hbm, ad_hbm,
                        logit_hbm, tmax_hbm,
                        srcv0, srcv1, dstv0, dstv1, lgv0, lgv1,
                        asv, adv, tmaxv, sin0, sin1, so0, so1):
    w = _wid()
    base = w * EPT
    srcv = (srcv0, srcv1)
    dstv = (dstv0, dstv1)
    lgv = (lgv0, lgv1)
    sin = (sin0, sin1)
    so = (so0, so1)
    pltpu.sync_copy(as_hbm, asv)
    pltpu.sync_copy(ad_hbm, adv)
    tmaxv[...] = jnp.full((16,), -jnp.inf, jnp.float32)

    def fire_in(k2, b):
        cb = base + k2 * CK2
        pltpu.async_copy(src_hbm.at[pl.ds(cb, CK2)], srcv[b].at[0], sin[b])
        pltpu.async_copy(dst_hbm.at[pl.ds(cb, CK2)], dstv[b].at[0], sin[b])

    def wait_in(k2, b):
        cb = base + k2 * CK2
        pltpu.make_async_copy(src_hbm.at[pl.ds(cb, CK2)], srcv[b].at[0],
                              sin[b]).wait()
        pltpu.make_async_copy(dst_hbm.at[pl.ds(cb, CK2)], dstv[b].at[0],
                              sin[b]).wait()

    def wait_out(k2, b):
        cb = base + k2 * CK2
        pltpu.make_async_copy(lgv[b], logit_hbm.at[pl.ds(cb, CK2)],
                              so[b]).wait()

    fire_in(0, 0)

    def pair(gi, _):
        for b in (0, 1):
            k2 = gi * 2 + b
            wait_in(k2, b)

            @pl.when(k2 >= 2)
            def _():
                wait_out(k2 - 2, b)

            @pl.when(k2 + 1 < NCHUNK2)
            def _():
                fire_in(k2 + 1, 1 - b)

            def group(g, _):
                sl = pl.ds(g * 16, 16)
                sa = plsc.load_gather(asv, [srcv[b][0, sl]])
                da = plsc.load_gather(adv, [dstv[b][0, sl]])
                lg = _lrelu(sa + da)
                lgv[b][sl] = lg
                tmaxv[...] = jnp.maximum(tmaxv[...], lg)
                return 0

            lax.fori_loop(0, CK2 // 16, group, 0)
            cb = base + k2 * CK2
            pltpu.async_copy(lgv[b], logit_hbm.at[pl.ds(cb, CK2)], so[b])
        return 0

    lax.fori_loop(0, NCHUNK2 // 2, pair, 0)
    wait_out(NCHUNK2 - 2, 0)
    wait_out(NCHUNK2 - 1, 1)
    pltpu.sync_copy(tmaxv, tmax_hbm.at[w])


def _sc_gat_logits(src_p, dst_p, asrc_n, adst_n):
    return pl.kernel(
        _sc_gat_logits_body,
        out_type=[
            jax.ShapeDtypeStruct((EP,), jnp.float32),
            jax.ShapeDtypeStruct((NW, 16), jnp.float32),
        ],
        mesh=_sc_mesh(),
        scratch_types=(
            [pltpu.VMEM((1, CK2), jnp.int32) for _ in range(4)]
            + [pltpu.VMEM((CK2,), jnp.float32) for _ in range(2)]
            + [pltpu.VMEM((NP,), jnp.float32),
               pltpu.VMEM((NP,), jnp.float32),
               pltpu.VMEM((16,), jnp.float32)]
            + [pltpu.SemaphoreType.DMA for _ in range(4)]
        ),
        compiler_params=pltpu.CompilerParams(needs_layout_passes=False),
    )(src_p, dst_p, asrc_n, adst_n)


CK2 = 64
NCHUNK2 = EPT // CK2


def _sc_scatter_body(src_hbm, dst_hbm, tab_hbm, ew_hbm, lg_hbm, tmax_hbm,
                     zrow_hbm, zvec_hbm, acc_hbm, s_hbm,
                     srcv0, srcv1, dstv0, dstv1, dsts0, dsts1,
                     lgv0, lgv1, ebuf0, ebuf1, ewv0, ewv1, gatv0, gatv1,
                     tmaxs, gmaxv, acc_sh, s_sh,
                     sin0, sin1, sg0, sg1, so0, so1, *, gate):
    c = lax.axis_index("c")
    sid = lax.axis_index("s")
    base = (sid * NC + c) * EPT
    srcv = (srcv0, srcv1)
    dstv = (dstv0, dstv1)
    dsts = (dsts0, dsts1)
    lgv = (lgv0, lgv1)
    ebuf = (ebuf0, ebuf1)
    ewv = (ewv0, ewv1)
    gatv = (gatv0, gatv1)
    sin = (sin0, sin1)
    sg = (sg0, sg1)
    so = (so0, so1)

    pltpu.sync_copy(zrow_hbm, acc_sh.at[pl.ds(sid * ROWS, ROWS), :])
    pltpu.sync_copy(zvec_hbm, s_sh.at[pl.ds(sid * ROWS, ROWS)])
    pltpu.sync_copy(tmax_hbm, tmaxs)
    gm = jnp.full((16,), -jnp.inf, jnp.float32)
    for t in range(NW):
        gm = jnp.maximum(gm, tmaxs[t, :])
    gmaxv[...] = jnp.full((16,), 0.0, jnp.float32) + jnp.max(gm)
    plsc.subcore_barrier()

    def fire_in(k2, b):
        cb = base + k2 * CK2
        pltpu.async_copy(src_hbm.at[pl.ds(cb, CK2)], srcv[b].at[0], sin[b])
        pltpu.async_copy(dst_hbm.at[pl.ds(cb, CK2)], dstv[b].at[0], sin[b])
        pltpu.async_copy(lg_hbm.at[pl.ds(cb, CK2)], lgv[b], sin[b])
        if gate:
            pltpu.async_copy(ew_hbm.at[pl.ds(cb, CK2), :], ewv[b], sin[b])

    def wait_in(k2, b):
        cb = base + k2 * CK2
        pltpu.make_async_copy(src_hbm.at[pl.ds(cb, CK2)], srcv[b].at[0],
                              sin[b]).wait()
        pltpu.make_async_copy(dst_hbm.at[pl.ds(cb, CK2)], dstv[b].at[0],
                              sin[b]).wait()
        pltpu.make_async_copy(lg_hbm.at[pl.ds(cb, CK2)], lgv[b],
                              sin[b]).wait()
        if gate:
            pltpu.make_async_copy(ew_hbm.at[pl.ds(cb, CK2), :], ewv[b],
                                  sin[b]).wait()

    def wait_scatter(b):
        pltpu.make_async_copy(gatv[b], acc_sh.at[dsts[b].at[0]],
                              so[b]).wait()
        pltpu.make_async_copy(ebuf[b], s_sh.at[dsts[b].at[0]],
                              so[b]).wait()

    fire_in(0, 0)

    def pair(gi, _):
        for b in (0, 1):
            k2 = gi * 2 + b
            wait_in(k2, b)

            @pl.when(k2 >= 2)
            def _():
                wait_scatter(b)

            pltpu.async_copy(tab_hbm.at[srcv[b].at[0]], gatv[b], sg[b])

            @pl.when(k2 + 1 < NCHUNK2)
            def _():
                fire_in(k2 + 1, 1 - b)

            pltpu.make_async_copy(tab_hbm.at[srcv[b].at[0]], gatv[b],
                                  sg[b]).wait()
            gmax = gmaxv[...]
            for g in range(CK2 // 16):
                sl = pl.ds(g * 16, 16)
                ebuf[b][sl] = jnp.exp(lgv[b][sl] - gmax)
                dsts[b][0, sl] = dstv[b][0, sl]

            def edge(j, _):
                eb = _bcast16(ebuf[b], j)
                for v in range(8):
                    sl = pl.ds(v * 16, 16)
                    if gate:
                        row = _lrelu(gatv[b][j, sl] + ewv[b][j, sl])
                    else:
                        row = gatv[b][j, sl]
                    gatv[b][j, sl] = row * eb
                return 0

            lax.fori_loop(0, CK2, edge, 0)
            pltpu.async_copy(gatv[b], acc_sh.at[dsts[b].at[0]], so[b],
                             add=True)
            pltpu.async_copy(ebuf[b], s_sh.at[dsts[b].at[0]], so[b],
                             add=True)
        return 0

    lax.fori_loop(0, NCHUNK2 // 2, pair, 0)
    wait_scatter(0)
    wait_scatter(1)
    plsc.subcore_barrier()
    pltpu.sync_copy(acc_sh.at[pl.ds(sid * ROWS, ROWS), :],
                    acc_hbm.at[c, pl.ds(sid * ROWS, ROWS), :])
    pltpu.sync_copy(s_sh.at[pl.ds(sid * ROWS, ROWS)],
                    s_hbm.at[c, pl.ds(sid * ROWS, ROWS)])


def _sc_scatter(src_p, dst_p, table, ew, logit, tmax, zrow, zvec, gate):
    body = functools.partial(_sc_scatter_body, gate=gate)
    return pl.kernel(
        body,
        out_type=[
            jax.ShapeDtypeStruct((NC, NP, H), jnp.float32),
            jax.ShapeDtypeStruct((NC, NP), jnp.float32),
        ],
        mesh=_sc_mesh(),
        scratch_types=(
            [pltpu.VMEM((1, CK2), jnp.int32) for _ in range(6)]
            + [pltpu.VMEM((CK2,), jnp.float32) for _ in range(4)]
            + [pltpu.VMEM((CK2, H), jnp.float32) for _ in range(4)]
            + [pltpu.VMEM((NW, 16), jnp.float32),
               pltpu.VMEM((16,), jnp.float32),
               pltpu.VMEM_SHARED((NP, H), jnp.float32),
               pltpu.VMEM_SHARED((NP,), jnp.float32)]
            + [pltpu.SemaphoreType.DMA for _ in range(6)]
        ),
        compiler_params=pltpu.CompilerParams(needs_layout_passes=False),
    )(src_p, dst_p, table, ew, logit, tmax, zrow, zvec)


def kernel(x, edge_index, edge_attr, batch, params):
    p = params
    x_p = jnp.zeros((NP, DIN), jnp.float32).at[:N].set(x)
    src_p = jnp.full((EP,), N, jnp.int32).at[:E].set(edge_index[0])
    dst_p = jnp.full((EP,), N, jnp.int32).at[:E].set(edge_index[1])
    ea_p = jnp.zeros((EP, DEDGE), jnp.float32).at[:E].set(edge_attr)
    batch_p = jnp.full((NP,), G, jnp.int32).at[:N].set(batch)
    zrow = jnp.zeros((ROWS, H), jnp.float32)
    zvec = jnp.zeros((ROWS,), jnp.float32)
    Wg1x = p['Wg1'][:, :H]
    Wg1e = p['Wg1'][:, H:]

    x1, xw1, ar = _tc1(x_p, p['W_lin1'], p['b_lin1'], Wg1x, p['att_r'])
    ew = _tc2(ea_p, Wg1e)

    logit, tmax = _sc_gate_logits(src_p, dst_p, ew, xw1, ar, p['att_l'])
    acc, s = _sc_scatter(src_p, dst_p, xw1, ew, logit, tmax, zrow, zvec, True)
    x2, hx1, as1, ad1 = _tc_update(
        acc, s, x1, p['Wg2'], p['b_gate'], p['Wi0'], p['Wh0'], p['bi0'],
        p['bh0'], p['Wa1'], p['asrc1'], p['adst1'], True)

    logit, tmax = _sc_gat_logits(src_p, dst_p, as1, ad1)
    acc, s = _sc_scatter(src_p, dst_p, hx1, ew, logit, tmax, zrow, zvec,
                         False)
    x3, hx2, as2, ad2 = _tc_update(
        acc, s, x2, p['Wg2'], p['ba1'], p['Wi1'], p['Wh1'], p['bi1'],
        p['bh1'], p['Wa2'], p['asrc2'], p['adst2'], False)

    logit, tmax = _sc_gat_logits(src_p, dst_p, as2, ad2)
    acc, s = _sc_scatter(src_p, dst_p, hx2, ew, logit, tmax, zrow, zvec,
                         False)

    hsrc, ansrc, out_g = _tc5(
        acc, s, x3, p['ba2'], p['Wi2'], p['Wh2'], p['bi2'], p['bh2'],
        p['Wm'], p['asrcm'], batch_p)

    for _ in range(3):
        out_g = _mc_iter(hsrc, ansrc, batch_p, out_g, p['Wm'], p['adstm'],
                         p['bm'], p['Wim'], p['Whm'], p['bim'], p['bhm'])

    return _tc6(out_g, p['W_lin2'], p['b_lin2'], p['W_mlp1'], p['b_mlp1'],
                p['W_mlp2'], p['b_mlp2'])
